# Initial kernel scaffold; baseline (speedup 1.0000x reference)
import jax
import jax.numpy as jnp
from jax.experimental import pallas as pl


def kernel(query_key, keys, usage, k):
    raise NotImplementedError("write your pallas kernel here")


# speedup vs baseline: 1.8012x; 1.8012x over previous
import functools

import jax
import jax.numpy as jnp
from jax import lax
from jax.experimental import pallas as pl
from jax.experimental.pallas import tpu as pltpu
from jax.experimental.pallas import tpu_sc as plsc

CAP = 1_000_000
DIM = 64
K = 32
BLK = 10_000
STEPS = CAP // BLK

def _tc_body(q_ref, keys_ref, dist_ref, idx_ref, d2_ref):
    i = pl.program_id(0)
    q = q_ref[...]
    x = keys_ref[...]
    diff = x - q
    sq = diff * diff
    ones = jnp.ones((1, DIM), jnp.float32)
    d2 = lax.dot_general(ones, sq, (((1,), (1,)), ((), ())),
                         precision=lax.Precision.HIGHEST,
                         preferred_element_type=jnp.float32)
    d2_ref[pl.ds(i, 1), :] = d2

    @pl.when(i == STEPS - 1)
    def _final():
        all_d2 = d2_ref[...]
        inf = jnp.float32(jnp.inf)
        big = jnp.int32(2**30)
        cols = lax.broadcasted_iota(jnp.int32, (1, BLK), 1)

        cm = jnp.min(all_d2, axis=0, keepdims=True)
        sel = []
        for _ in range(K):
            m = jnp.min(cm)
            c = jnp.min(jnp.where(cm == m, cols, big))
            sel.append(c)
            cm = jnp.where(cols == c, inf, cm)

        oh = jnp.concatenate([(cols == c).astype(jnp.float32) for c in sel],
                             axis=0)
        cand = lax.dot_general(all_d2, oh, (((1,), (1,)), ((), ())),
                               precision=lax.Precision.HIGHEST,
                               preferred_element_type=jnp.float32)

        cvec = jnp.concatenate([c.reshape(1, 1) for c in sel], axis=1)
        rows = lax.broadcasted_iota(jnp.int32, (STEPS, K), 0)
        gmat = rows * BLK + cvec
        kiota = lax.broadcasted_iota(jnp.int32, (1, K), 1)
        dvec = jnp.zeros((1, K), jnp.float32)
        ivec = jnp.zeros((1, K), jnp.int32)
        for t in range(K):
            m = jnp.min(cand)
            g = jnp.min(jnp.where(cand == m, gmat, big))
            dvec = jnp.where(kiota == t, m, dvec)
            ivec = jnp.where(kiota == t, g, ivec)
            cand = jnp.where((cand == m) & (gmat == g), inf, cand)
        dist_ref[...] = jnp.sqrt(dvec)
        idx_ref[...] = ivec


_topk_call = pl.pallas_call(
    _tc_body,
    grid=(STEPS,),
    in_specs=[pl.BlockSpec((1, DIM), lambda i: (0, 0)),
              pl.BlockSpec((BLK, DIM), lambda i: (i, 0))],
    out_specs=[pl.BlockSpec((1, K), lambda i: (0, 0)),
               pl.BlockSpec((1, K), lambda i: (0, 0))],
    out_shape=[jax.ShapeDtypeStruct((1, K), jnp.float32),
               jax.ShapeDtypeStruct((1, K), jnp.int32)],
    scratch_shapes=[pltpu.VMEM((STEPS, BLK), jnp.float32)],
    compiler_params=pltpu.CompilerParams(
        dimension_semantics=("arbitrary",)),
)


_NW = 32
_CHUNK = 31_256
_LAST = CAP - (_NW - 1) * _CHUNK


def _usage_apply(buf, idxv, base, size):
    ones16 = jnp.ones((16,), jnp.float32)
    for h in range(K // 16):
        iv = idxv[pl.ds(h * 16, 16)]
        m = (iv >= base) & (iv < base + size)
        loc = jnp.where(m, iv - base, 0)
        plsc.addupdate_scatter(buf, [loc], ones16, mask=m)


def _usage_body(usage_hbm, idx_hbm, out_hbm, buf, idxv):
    c = lax.axis_index("c")
    s = lax.axis_index("s")
    wid = s * 2 + c
    base = wid * _CHUNK
    pltpu.sync_copy(idx_hbm, idxv)

    @pl.when(wid < _NW - 1)
    def _main():
        pltpu.sync_copy(usage_hbm.at[pl.ds(base, _CHUNK)], buf)
        _usage_apply(buf, idxv, base, _CHUNK)
        pltpu.sync_copy(buf, out_hbm.at[pl.ds(base, _CHUNK)])

    @pl.when(wid == _NW - 1)
    def _tail():
        pltpu.sync_copy(usage_hbm.at[pl.ds(base, _LAST)],
                        buf.at[pl.ds(0, _LAST)])
        _usage_apply(buf, idxv, base, _LAST)
        pltpu.sync_copy(buf.at[pl.ds(0, _LAST)],
                        out_hbm.at[pl.ds(base, _LAST)])


@functools.cache
def _usage_call():
    return functools.partial(
        pl.kernel,
        out_type=jax.ShapeDtypeStruct((CAP,), jnp.float32),
        mesh=plsc.VectorSubcoreMesh(core_axis_name="c", subcore_axis_name="s"),
        scratch_types=[pltpu.VMEM((_CHUNK,), jnp.float32),
                       pltpu.VMEM((K,), jnp.int32)],
        compiler_params=pltpu.CompilerParams(needs_layout_passes=False),
    )(_usage_body)


def kernel(query_key, keys, usage, k):
    q2 = query_key.reshape(1, DIM)
    dist2, idx2 = _topk_call(q2, keys)
    topk_d = dist2.reshape(K)
    topk_i = idx2.reshape(K)
    new_usage = _usage_call()(usage, topk_i)
    return topk_d, topk_i, new_usage
